# baseline (speedup 1.0000x reference)
import functools

import jax
import jax.numpy as jnp
from jax import lax
from jax.experimental import pallas as pl
from jax.experimental.pallas import tpu as pltpu
from jax.experimental.pallas import tpu_sc as plsc

D_MODEL = 64
SAMPLE = 200
BATCH = 4096
LANES = 16
NUM_CORES = 2
NUM_SUBCORES = 16
NUM_WORKERS = NUM_CORES * NUM_SUBCORES
NB = BATCH // NUM_WORKERS
CHUNK = 100
GROUPS = 13
SPAD = GROUPS * LANES


def _sc_body(h_hbm, idx_hbm, tbl_hbm, out_hbm,
             h_v, idx_v, rows0, rows1, st0, st1,
             sg0, sg1, so0, so1):
    wid = lax.axis_index("s") * NUM_CORES + lax.axis_index("c")
    b0 = wid * NB

    pltpu.sync_copy(h_hbm.at[pl.ds(b0 * D_MODEL, NB * D_MODEL)], h_v)
    pltpu.sync_copy(idx_hbm.at[pl.ds(wid * (2 * NB), 2 * NB)], idx_v)

    iota = lax.iota(jnp.int32, LANES)
    rowidx = [iota + LANES * g for g in range(GROUPS)]

    def gather_cps(bl, rows, sem):
        return [
            pltpu.make_async_copy(
                tbl_hbm.at[idx_v.at[2 * bl + c]],
                rows.at[pl.ds(CHUNK * c, CHUNK)],
                sem,
            )
            for c in range(2)
        ]

    def fire(bl, rows, sem):
        for cp in gather_cps(bl, rows, sem):
            cp.start()

    def wait(bl, rows, sem):
        for cp in gather_cps(bl, rows, sem):
            cp.wait()

    def compute(bl, rows, stage, semo):
        def dbody(d, accs):
            colv = jnp.full((LANES,), d, jnp.int32)
            hb = plsc.load_gather(h_v, [jnp.full((LANES,), bl * D_MODEL + d, jnp.int32)])
            return tuple(
                acc + hb * plsc.load_gather(rows, [rowidx[g], colv])
                for g, acc in enumerate(accs)
            )

        zero = jnp.zeros((LANES,), jnp.float32)
        accs = lax.fori_loop(0, D_MODEL, dbody, (zero,) * GROUPS)

        out_row = out_hbm.at[b0 + bl]
        @pl.when(bl >= 2)
        def _():
            pltpu.make_async_copy(stage.at[pl.ds(0, SAMPLE)], out_row, semo).wait()

        for g in range(GROUPS):
            stage[pl.ds(LANES * g, LANES)] = accs[g]
        pltpu.make_async_copy(stage.at[pl.ds(0, SAMPLE)], out_row, semo).start()

    fire(0, rows0, sg0)

    def pair(i, carry):
        a = 2 * i
        fire(a + 1, rows1, sg1)
        wait(a, rows0, sg0)
        compute(a, rows0, st0, so0)

        @pl.when(i < NB // 2 - 1)
        def _():
            fire(a + 2, rows0, sg0)

        wait(a + 1, rows1, sg1)
        compute(a + 1, rows1, st1, so1)
        return carry

    lax.fori_loop(0, NB // 2, pair, 0)

    pltpu.make_async_copy(st0.at[pl.ds(0, SAMPLE)], out_hbm.at[b0 + NB - 2], so0).wait()
    pltpu.make_async_copy(st1.at[pl.ds(0, SAMPLE)], out_hbm.at[b0 + NB - 1], so1).wait()


@jax.jit
def _embedding_dot(h2, idx2, table):
    mesh = plsc.VectorSubcoreMesh(
        core_axis_name="c", subcore_axis_name="s",
        num_cores=NUM_CORES, num_subcores=NUM_SUBCORES,
    )
    call = functools.partial(
        pl.kernel,
        out_type=jax.ShapeDtypeStruct((BATCH, SAMPLE), jnp.float32),
        mesh=mesh,
        scratch_types=[
            pltpu.VMEM((NB * D_MODEL,), jnp.float32),
            pltpu.VMEM((2 * NB, CHUNK), jnp.int32),
            pltpu.VMEM((SPAD, D_MODEL), jnp.float32),
            pltpu.VMEM((SPAD, D_MODEL), jnp.float32),
            pltpu.VMEM((SPAD,), jnp.float32),
            pltpu.VMEM((SPAD,), jnp.float32),
            pltpu.SemaphoreType.DMA,
            pltpu.SemaphoreType.DMA,
            pltpu.SemaphoreType.DMA,
            pltpu.SemaphoreType.DMA,
        ],
        compiler_params=pltpu.CompilerParams(
            needs_layout_passes=False, use_tc_tiling_on_sc=False
        ),
    )
    return call(_sc_body)(h2, idx2, table)


def kernel(h, indicies, embedding_weight):
    b, s = indicies.shape
    h2 = jnp.reshape(h, (b * D_MODEL,))
    idx2 = jnp.reshape(indicies.astype(jnp.int32), (2 * b, CHUNK))
    out = _embedding_dot(h2, idx2, embedding_weight)
    return jnp.reshape(out, (b, 1, s))


# speedup vs baseline: 1.0345x; 1.0345x over previous
import functools

import jax
import jax.numpy as jnp
from jax import lax
from jax.experimental import pallas as pl
from jax.experimental.pallas import tpu as pltpu
from jax.experimental.pallas import tpu_sc as plsc

D_MODEL = 64
SAMPLE = 200
BATCH = 4096
LANES = 16
NUM_CORES = 2
NUM_SUBCORES = 16
NUM_WORKERS = NUM_CORES * NUM_SUBCORES
NB = BATCH // NUM_WORKERS
CHUNK = 100
GROUPS = 13
BPB = 2
ROWS = BPB * SAMPLE + 8
UNROLL = 4


def _sc_body(h_hbm, idx_hbm, tbl_hbm, out_hbm,
             h_v, idx_v, rows0, rows1, out_st, sg0, sg1):
    wid = lax.axis_index("s") * NUM_CORES + lax.axis_index("c")
    b0 = wid * NB

    pltpu.sync_copy(h_hbm.at[pl.ds(b0 * D_MODEL, NB * D_MODEL)], h_v)
    pltpu.sync_copy(idx_hbm.at[pl.ds(wid * (2 * NB), 2 * NB)], idx_v)

    iota = lax.iota(jnp.int32, LANES)
    rowidx = [
        [iota + slot * SAMPLE + LANES * g for g in range(GROUPS)]
        for slot in range(BPB)
    ]

    def gather_cps(b_first, rows, sem):
        return [
            pltpu.make_async_copy(
                tbl_hbm.at[idx_v.at[2 * b_first + c]],
                rows.at[pl.ds(CHUNK * c, CHUNK)],
                sem,
            )
            for c in range(2 * BPB)
        ]

    def fire(b_first, rows, sem):
        for cp in gather_cps(b_first, rows, sem):
            cp.start()

    def wait(b_first, rows, sem):
        for cp in gather_cps(b_first, rows, sem):
            cp.wait()

    def compute(bl, rows, slot):
        hbase = bl * D_MODEL

        def dbody(i, accs):
            for k in range(UNROLL):
                d = i * UNROLL + k
                colv = jnp.full((LANES,), d, jnp.int32)
                hb = plsc.load_gather(h_v, [jnp.full((LANES,), hbase + d, jnp.int32)])
                accs = tuple(
                    acc + hb * plsc.load_gather(rows, [rowidx[slot][g], colv])
                    for g, acc in enumerate(accs)
                )
            return accs

        zero = jnp.zeros((LANES,), jnp.float32)
        accs = lax.fori_loop(0, D_MODEL // UNROLL, dbody, (zero,) * GROUPS)

        obase = bl * SAMPLE
        for g in range(GROUPS):
            out_st[pl.ds(obase + LANES * g, LANES)] = accs[g]

    fire(0, rows0, sg0)

    def quad(i, carry):
        a = BPB * 2 * i
        fire(a + BPB, rows1, sg1)
        wait(a, rows0, sg0)
        compute(a + 0, rows0, 0)
        compute(a + 1, rows0, 1)

        @pl.when(i < NB // (2 * BPB) - 1)
        def _():
            fire(a + 2 * BPB, rows0, sg0)

        wait(a + BPB, rows1, sg1)
        compute(a + BPB + 0, rows1, 0)
        compute(a + BPB + 1, rows1, 1)
        return carry

    lax.fori_loop(0, NB // (2 * BPB), quad, 0)

    pltpu.sync_copy(
        out_st.at[pl.ds(0, NB * SAMPLE)],
        out_hbm.at[pl.ds(wid * NB * SAMPLE, NB * SAMPLE)],
    )


@jax.jit
def _embedding_dot(h2, idx2, table):
    mesh = plsc.VectorSubcoreMesh(
        core_axis_name="c", subcore_axis_name="s",
        num_cores=NUM_CORES, num_subcores=NUM_SUBCORES,
    )
    call = functools.partial(
        pl.kernel,
        out_type=jax.ShapeDtypeStruct((BATCH * SAMPLE,), jnp.float32),
        mesh=mesh,
        scratch_types=[
            pltpu.VMEM((NB * D_MODEL,), jnp.float32),
            pltpu.VMEM((2 * NB, CHUNK), jnp.int32),
            pltpu.VMEM((ROWS, D_MODEL), jnp.float32),
            pltpu.VMEM((ROWS, D_MODEL), jnp.float32),
            pltpu.VMEM((NB * SAMPLE + 8,), jnp.float32),
            pltpu.SemaphoreType.DMA,
            pltpu.SemaphoreType.DMA,
        ],
        compiler_params=pltpu.CompilerParams(
            needs_layout_passes=False, use_tc_tiling_on_sc=False
        ),
    )
    return call(_sc_body)(h2, idx2, table)


def kernel(h, indicies, embedding_weight):
    b, s = indicies.shape
    h2 = jnp.reshape(h, (b * D_MODEL,))
    idx2 = jnp.reshape(indicies.astype(jnp.int32), (2 * b, CHUNK))
    out = _embedding_dot(h2, idx2, embedding_weight)
    return jnp.reshape(out, (b, 1, s))


# speedup vs baseline: 2.0695x; 2.0006x over previous
import functools

import jax
import jax.numpy as jnp
from jax import lax
from jax.experimental import pallas as pl
from jax.experimental.pallas import tpu as pltpu
from jax.experimental.pallas import tpu_sc as plsc

D_MODEL = 64
SAMPLE = 200
BATCH = 4096
LANES = 16
NUM_CORES = 2
NUM_SUBCORES = 16
NUM_WORKERS = NUM_CORES * NUM_SUBCORES
NB = BATCH // NUM_WORKERS
CHUNK = 100
GROUPS = 13
BPB = 2
ROWS = BPB * SAMPLE + 8
UNROLL = 4


def _sc_body(h_hbm, idx_hbm, tbl_hbm, out_hbm,
             h_v, idx_v, rows0, rows1, out_st, sg0, sg1):
    wid = lax.axis_index("s") * NUM_CORES + lax.axis_index("c")
    b0 = wid * NB

    pltpu.sync_copy(h_hbm.at[pl.ds(b0 * D_MODEL, NB * D_MODEL)], h_v)
    pltpu.sync_copy(idx_hbm.at[pl.ds(wid * (2 * NB), 2 * NB)], idx_v)

    iota = lax.iota(jnp.int32, LANES)
    rowidx = [
        [iota + slot * SAMPLE + LANES * g for g in range(GROUPS)]
        for slot in range(BPB)
    ]

    def gather_cps(b_first, rows, sem):
        return [
            pltpu.make_async_copy(
                tbl_hbm.at[idx_v.at[2 * b_first + c]],
                rows.at[pl.ds(CHUNK * c, CHUNK)],
                sem,
            )
            for c in range(2 * BPB)
        ]

    def fire(b_first, rows, sem):
        for cp in gather_cps(b_first, rows, sem):
            cp.start()

    def wait(b_first, rows, sem):
        for cp in gather_cps(b_first, rows, sem):
            cp.wait()

    def compute(bl, rows, slot):
        hbase = bl * D_MODEL

        def dbody(i, accs):
            for k in range(UNROLL):
                d = i * UNROLL + k
                colv = (iota + d) & (D_MODEL - 1)
                hb = plsc.load_gather(h_v, [hbase + colv])
                accs = tuple(
                    acc + hb * plsc.load_gather(rows, [rowidx[slot][g], colv])
                    for g, acc in enumerate(accs)
                )
            return accs

        zero = jnp.zeros((LANES,), jnp.float32)
        accs = lax.fori_loop(0, D_MODEL // UNROLL, dbody, (zero,) * GROUPS)

        obase = bl * SAMPLE
        for g in range(GROUPS):
            out_st[pl.ds(obase + LANES * g, LANES)] = accs[g]

    fire(0, rows0, sg0)

    def quad(i, carry):
        a = BPB * 2 * i
        fire(a + BPB, rows1, sg1)
        wait(a, rows0, sg0)
        compute(a + 0, rows0, 0)
        compute(a + 1, rows0, 1)

        @pl.when(i < NB // (2 * BPB) - 1)
        def _():
            fire(a + 2 * BPB, rows0, sg0)

        wait(a + BPB, rows1, sg1)
        compute(a + BPB + 0, rows1, 0)
        compute(a + BPB + 1, rows1, 1)
        return carry

    lax.fori_loop(0, NB // (2 * BPB), quad, 0)

    pltpu.sync_copy(
        out_st.at[pl.ds(0, NB * SAMPLE)],
        out_hbm.at[pl.ds(wid * NB * SAMPLE, NB * SAMPLE)],
    )


@jax.jit
def _embedding_dot(h2, idx2, table):
    mesh = plsc.VectorSubcoreMesh(
        core_axis_name="c", subcore_axis_name="s",
        num_cores=NUM_CORES, num_subcores=NUM_SUBCORES,
    )
    call = functools.partial(
        pl.kernel,
        out_type=jax.ShapeDtypeStruct((BATCH * SAMPLE,), jnp.float32),
        mesh=mesh,
        scratch_types=[
            pltpu.VMEM((NB * D_MODEL,), jnp.float32),
            pltpu.VMEM((2 * NB, CHUNK), jnp.int32),
            pltpu.VMEM((ROWS, D_MODEL), jnp.float32),
            pltpu.VMEM((ROWS, D_MODEL), jnp.float32),
            pltpu.VMEM((NB * SAMPLE + 8,), jnp.float32),
            pltpu.SemaphoreType.DMA,
            pltpu.SemaphoreType.DMA,
        ],
        compiler_params=pltpu.CompilerParams(
            needs_layout_passes=False, use_tc_tiling_on_sc=False
        ),
    )
    return call(_sc_body)(h2, idx2, table)


def kernel(h, indicies, embedding_weight):
    b, s = indicies.shape
    h2 = jnp.reshape(h, (b * D_MODEL,))
    idx2 = jnp.reshape(indicies.astype(jnp.int32), (2 * b, CHUNK))
    out = _embedding_dot(h2, idx2, embedding_weight)
    return jnp.reshape(out, (b, 1, s))
